# Initial kernel scaffold; baseline (speedup 1.0000x reference)
import jax
import jax.numpy as jnp
from jax.experimental import pallas as pl


def kernel(h, x, edge_index, params):
    raise NotImplementedError("write your pallas kernel here")


# speedup vs baseline: 1.0717x; 1.0717x over previous
import functools

import jax
import jax.numpy as jnp
from jax.experimental import pallas as pl
from jax.experimental.pallas import tpu as pltpu

_D = 128
_TW = 144


def _silu(v):
    return v * jax.nn.sigmoid(v)


def _edge_body(gr_ref, gc_ref, w1a_ref, w1b_ref, w1c_ref, b1_ref,
               w2_ref, b2_ref, c1_ref, cb1_ref, c2_ref, cb2_ref,
               m_ref, s_ref, *, final: bool):
    hr = gr_ref[:, :_D]
    hc = gc_ref[:, :_D]
    xr = gr_ref[:, _D:_D + 8]
    xc = gc_ref[:, _D:_D + 8]
    rel = xr - xc
    d2 = jnp.sum(rel * rel, axis=1, keepdims=True)
    dist = jnp.sqrt(d2)
    lane = jax.lax.broadcasted_iota(jnp.int32, rel.shape, 1)
    sm = jnp.where(lane == 3, dist, rel)
    t = (jnp.dot(hr, w1a_ref[...], preferred_element_type=jnp.float32)
         + jnp.dot(hc, w1b_ref[...], preferred_element_type=jnp.float32)
         + jnp.dot(sm, w1c_ref[...], preferred_element_type=jnp.float32)
         + b1_ref[...])
    t = _silu(t)
    m = _silu(jnp.dot(t, w2_ref[...], preferred_element_type=jnp.float32)
              + b2_ref[...])
    u = _silu(jnp.dot(m, c1_ref[...], preferred_element_type=jnp.float32)
              + cb1_ref[...])
    v = jnp.dot(u, c2_ref[...], preferred_element_type=jnp.float32) + cb2_ref[...]
    if final:
        s = v
    else:
        cm = jnp.tanh(v[:, 0:1])
        w = cm / (dist + 1e-8)
        s = w * rel
    m_ref[...] = m
    s_ref[...] = s


def _edge_pass(gr, gc, wts, final, block=1600):
    e = gr.shape[0]
    assert e % block == 0
    grid = e // block
    full = lambda shape: pl.BlockSpec(shape, lambda i: (0, 0))
    blk = lambda w: pl.BlockSpec((block, w), lambda i: (i, 0))
    (w1a, w1b, w1c, b1, w2, b2, c1, cb1, c2, cb2) = wts
    m, s = pl.pallas_call(
        functools.partial(_edge_body, final=final),
        grid=(grid,),
        in_specs=[
            blk(_TW), blk(_TW),
            full((_D, _D)), full((_D, _D)), full((8, _D)), full((1, _D)),
            full((_D, _D)), full((1, _D)),
            full((_D, _D)), full((1, _D)),
            full((_D, 8)), full((1, 8)),
        ],
        out_specs=[blk(_D), blk(8)],
        out_shape=[jax.ShapeDtypeStruct((e, _D), jnp.float32),
                   jax.ShapeDtypeStruct((e, 8), jnp.float32)],
    )(gr, gc, w1a, w1b, w1c, b1, w2, b2, c1, cb1, c2, cb2)
    return m, s


def _layer_weights(lp):
    w1 = lp['msg1'][0]
    w1c = jnp.zeros((8, _D), jnp.float32).at[3].set(w1[2 * _D])
    c2 = jnp.zeros((_D, 8), jnp.float32).at[:, 0].set(lp['c2'][0][:, 0])
    return (w1[:_D], w1[_D:2 * _D], w1c, lp['msg1'][1][None],
            lp['msg2'][0], lp['msg2'][1][None],
            lp['c1'][0], lp['c1'][1][None],
            c2, jnp.zeros((1, 8), jnp.float32))


def _final_weights(p):
    w1 = p['em1'][0]
    w1c = jnp.zeros((8, _D), jnp.float32).at[:3].set(w1[2 * _D:2 * _D + 3])
    w1c = w1c.at[3].set(w1[2 * _D + 3])
    c2 = jnp.zeros((_D, 8), jnp.float32).at[:, :3].set(p['ec2'][0])
    cb2 = jnp.zeros((1, 8), jnp.float32).at[0, :3].set(p['ec2'][1])
    return (w1[:_D], w1[_D:2 * _D], w1c, p['em1'][1][None],
            p['em2'][0], p['em2'][1][None],
            p['ec1'][0], p['ec1'][1][None],
            c2, cb2)


def _ln(v, g, b):
    mu = jnp.mean(v, axis=-1, keepdims=True)
    var = jnp.var(v, axis=-1, keepdims=True)
    return (v - mu) / jnp.sqrt(var + 1e-5) * g + b


def _table(h, x):
    n = h.shape[0]
    pad = jnp.zeros((n, _TW - _D - 3), jnp.float32)
    t = jnp.concatenate([h, x, pad], axis=1)
    return jnp.concatenate([t, jnp.zeros((1, _TW), jnp.float32)], axis=0)


def kernel(h, x, edge_index, params):
    p = params
    n, e = h.shape[0], edge_index.shape[1]
    block = 1600
    ep = -(-e // block) * block
    row = jnp.concatenate([edge_index[0], jnp.full((ep - e,), n, jnp.int32)])
    col = jnp.concatenate([edge_index[1], jnp.full((ep - e,), n, jnp.int32)])
    colv = edge_index[1]

    wemb, bemb = p['emb']
    h = h @ wemb + bemb
    deg = jnp.zeros((n,), jnp.float32).at[colv].add(1.0)[:, None]

    for lp in p['layers']:
        t = _table(h, x)
        m, s = _edge_pass(jnp.take(t, row, axis=0), jnp.take(t, col, axis=0),
                          _layer_weights(lp), final=False, block=block)
        agg = jnp.zeros((n, _D), jnp.float32).at[colv].add(m[:e])
        cu = jnp.zeros((n, 3), jnp.float32).at[colv].add(s[:e, :3])
        x = x + cu / (deg + 1.0)
        u1, ub1 = lp['u1']
        u2, ub2 = lp['u2']
        hu = _silu(h @ u1[:_D] + agg @ u1[_D:] + ub1) @ u2 + ub2
        g, bb = lp['ln']
        h = _ln(h + hu, g, bb)

    t = _table(h, x)
    _, s = _edge_pass(jnp.take(t, row, axis=0), jnp.take(t, col, axis=0),
                      _final_weights(p), final=True, block=block)
    eps = jnp.zeros((n, 3), jnp.float32).at[colv].add(s[:e, :3])
    hw1, hb1 = p['eh1']
    hw2, hb2 = p['eh2']
    eps = eps + _silu(h @ hw1[:_D] + x @ hw1[_D:] + hb1) @ hw2 + hb2
    return (h, x, eps)
